# baseline (speedup 1.0000x reference)
import jax
import jax.numpy as jnp
from jax import lax
from jax.experimental import pallas as pl
from jax.experimental.pallas import tpu as pltpu
from jax.experimental.pallas import tpu_sc as plsc

N_PROMPT = 170
SEQ = 512
TOK = SEQ - N_PROMPT
BSZ = 1024
D = 128
NC, NS = 2, 16
NW = NC * NS
B_PER_W = BSZ // NW
IDXROW = 344


def _body(idx_hbm, table_hbm, pna, p1, p2, p3, p4, p5, psep, out_hbm,
          idx_v, prompt_v, bufs, gsems, wsems, psem):
    c = lax.axis_index("c")
    s = lax.axis_index("s")
    wid = s * NC + c
    b0 = wid * B_PER_W

    pltpu.sync_copy(pna, prompt_v.at[pl.ds(0, 1)])
    pltpu.sync_copy(p1, prompt_v.at[pl.ds(1, 34)])
    pltpu.sync_copy(p2, prompt_v.at[pl.ds(35, 34)])
    pltpu.sync_copy(p3, prompt_v.at[pl.ds(69, 34)])
    pltpu.sync_copy(p4, prompt_v.at[pl.ds(103, 33)])
    pltpu.sync_copy(p5, prompt_v.at[pl.ds(136, 33)])
    pltpu.sync_copy(psep, prompt_v.at[pl.ds(169, 1)])

    pltpu.sync_copy(idx_hbm.at[pl.ds(b0, B_PER_W)], idx_v)

    def fire_gather(i, r):
        pltpu.async_copy(table_hbm.at[idx_v.at[i]], bufs[r], gsems[r])

    def gwait(i, r):
        pltpu.make_async_copy(table_hbm.at[idx_v.at[i]],
                              bufs[r], gsems[r]).wait()

    def fire_writes(i, r):
        b = b0 + i
        pltpu.async_copy(bufs[r].at[pl.ds(0, TOK)],
                         out_hbm.at[pl.ds(b * SEQ, TOK)], wsems[r])

    def wait_writes(i, r):
        b = b0 + i
        pltpu.make_async_copy(bufs[r].at[pl.ds(0, TOK)],
                              out_hbm.at[pl.ds(b * SEQ, TOK)],
                              wsems[r]).wait()

    fire_gather(0, 0)
    fire_gather(1, 1)

    def round_body(t, last):
        i = 2 * t
        gwait(i, 0)
        gwait(i + 1, 1)
        fire_writes(i, 0)
        fire_writes(i + 1, 1)
        if not last:
            wait_writes(i, 0)
            fire_gather(i + 2, 0)
            wait_writes(i + 1, 1)
            fire_gather(i + 3, 1)
        return 0

    lax.fori_loop(0, B_PER_W // 2 - 1,
                  lambda t, u: round_body(t, False), 0)
    round_body(B_PER_W // 2 - 1, True)
    wait_writes(B_PER_W - 2, 0)
    wait_writes(B_PER_W - 1, 1)

    def fire_prompt(i, u):
        pltpu.async_copy(
            prompt_v, out_hbm.at[pl.ds((b0 + i) * SEQ + TOK, N_PROMPT)],
            psem)
        return u

    def drain_prompt(i, u):
        pltpu.make_async_copy(
            prompt_v, out_hbm.at[pl.ds((b0 + i) * SEQ + TOK, N_PROMPT)],
            psem).wait()
        return u

    lax.fori_loop(0, B_PER_W, fire_prompt, 0)
    lax.fori_loop(0, B_PER_W, drain_prompt, 0)


_sc_call = pl.kernel(
    _body,
    out_type=jax.ShapeDtypeStruct((BSZ * SEQ, D), jnp.float32),
    mesh=plsc.VectorSubcoreMesh(
        core_axis_name="c", subcore_axis_name="s",
        num_cores=NC, num_subcores=NS,
    ),
    scratch_types=[
        pltpu.VMEM((B_PER_W, IDXROW), jnp.int32),
        pltpu.VMEM((N_PROMPT, D), jnp.float32),
        [pltpu.VMEM((IDXROW, D), jnp.float32)] * 2,
        [pltpu.SemaphoreType.DMA] * 2,
        [pltpu.SemaphoreType.DMA] * 2,
        pltpu.SemaphoreType.DMA,
    ],
    compiler_params=pltpu.CompilerParams(use_tc_tiling_on_sc=False),
)


@jax.jit
def kernel(tokens, embed_table, prompt_na, prompt1, prompt2, prompt3,
           prompt4, prompt5, prompt_sep):
    idx = jnp.pad(tokens[:, :TOK], ((0, 0), (0, IDXROW - TOK)))
    out = _sc_call(idx, embed_table, prompt_na, prompt1, prompt2, prompt3,
                   prompt4, prompt5, prompt_sep)
    return out.reshape(BSZ, SEQ, D)


# speedup vs baseline: 1.0128x; 1.0128x over previous
import jax
import jax.numpy as jnp
from jax import lax
from jax.experimental import pallas as pl
from jax.experimental.pallas import tpu as pltpu
from jax.experimental.pallas import tpu_sc as plsc

N_PROMPT = 170
SEQ = 512
TOK = SEQ - N_PROMPT
BSZ = 1024
D = 128
NC, NS = 2, 16
NW = NC * NS
B_PER_W = BSZ // NW
IDXROW = 344


def _body(idx_hbm, table_hbm, pna, p1, p2, p3, p4, p5, psep, out_hbm,
          idx_v, prompt_v, bufs, gsems, wsems):
    c = lax.axis_index("c")
    s = lax.axis_index("s")
    wid = s * NC + c
    b0 = wid * B_PER_W

    pltpu.sync_copy(pna, prompt_v.at[pl.ds(0, 1)])
    pltpu.sync_copy(p1, prompt_v.at[pl.ds(1, 34)])
    pltpu.sync_copy(p2, prompt_v.at[pl.ds(35, 34)])
    pltpu.sync_copy(p3, prompt_v.at[pl.ds(69, 34)])
    pltpu.sync_copy(p4, prompt_v.at[pl.ds(103, 33)])
    pltpu.sync_copy(p5, prompt_v.at[pl.ds(136, 33)])
    pltpu.sync_copy(psep, prompt_v.at[pl.ds(169, 1)])

    pltpu.sync_copy(idx_hbm.at[pl.ds(b0, B_PER_W)], idx_v)

    def fire_gather(i, r):
        pltpu.async_copy(table_hbm.at[idx_v.at[i]], bufs[r], gsems[r])

    def gwait(i, r):
        pltpu.make_async_copy(table_hbm.at[idx_v.at[i]],
                              bufs[r], gsems[r]).wait()

    def fire_writes(i, r):
        b = b0 + i
        pltpu.async_copy(bufs[r].at[pl.ds(0, TOK)],
                         out_hbm.at[pl.ds(b * SEQ, TOK)], wsems[r])
        pltpu.async_copy(prompt_v,
                         out_hbm.at[pl.ds(b * SEQ + TOK, N_PROMPT)],
                         wsems[r])

    def wait_writes(i, r):
        b = b0 + i
        pltpu.make_async_copy(bufs[r].at[pl.ds(0, TOK)],
                              out_hbm.at[pl.ds(b * SEQ, TOK)],
                              wsems[r]).wait()
        pltpu.make_async_copy(prompt_v,
                              out_hbm.at[pl.ds(b * SEQ + TOK, N_PROMPT)],
                              wsems[r]).wait()

    fire_gather(0, 0)
    fire_gather(1, 1)

    def round_body(t, last):
        for r in range(2):
            i = 2 * t + r
            gwait(i, r)
            fire_writes(i, r)
            if not last:
                wait_writes(i, r)
                fire_gather(i + 2, r)
        return 0

    lax.fori_loop(0, B_PER_W // 2 - 1,
                  lambda t, u: round_body(t, False), 0)
    round_body(B_PER_W // 2 - 1, True)
    wait_writes(B_PER_W - 2, 0)
    wait_writes(B_PER_W - 1, 1)


_sc_call = pl.kernel(
    _body,
    out_type=jax.ShapeDtypeStruct((BSZ * SEQ, D), jnp.float32),
    mesh=plsc.VectorSubcoreMesh(
        core_axis_name="c", subcore_axis_name="s",
        num_cores=NC, num_subcores=NS,
    ),
    scratch_types=[
        pltpu.VMEM((B_PER_W, IDXROW), jnp.int32),
        pltpu.VMEM((N_PROMPT, D), jnp.float32),
        [pltpu.VMEM((IDXROW, D), jnp.float32)] * 2,
        [pltpu.SemaphoreType.DMA] * 2,
        [pltpu.SemaphoreType.DMA] * 2,
    ],
    compiler_params=pltpu.CompilerParams(use_tc_tiling_on_sc=False),
)


@jax.jit
def kernel(tokens, embed_table, prompt_na, prompt1, prompt2, prompt3,
           prompt4, prompt5, prompt_sep):
    idx = jnp.pad(tokens[:, :TOK], ((0, 0), (0, IDXROW - TOK)))
    out = _sc_call(idx, embed_table, prompt_na, prompt1, prompt2, prompt3,
                   prompt4, prompt5, prompt_sep)
    return out.reshape(BSZ, SEQ, D)
